# baseline (speedup 1.0000x reference)
import jax
import jax.numpy as jnp
from jax import lax
from jax.experimental import pallas as pl
from jax.experimental.pallas import tpu as pltpu
from jax.experimental.pallas import tpu_sc as plsc

NUM_ENTITIES = 1000000
NUM_RELATIONS = 1000
EMBED_DIM = 64
BATCH = 16384

NC = 2
NS = 16
NW = NC * NS
B_PER_W = BATCH // NW
CHUNK = 128
NCHUNK = B_PER_W // CHUNK
L = 16


def _body(head_r, rel_r, tail_r, er, ei, rrt, rit, out_hbm,
          idx_h, idx_r, idx_t, g_hr, g_hi, g_tr, g_ti, g_rr, g_ri,
          out_v, sem):
    wid = lax.axis_index("s") * NC + lax.axis_index("c")
    iota = lax.iota(jnp.int32, L)

    def chunk_body(ci, carry):
        base = wid * B_PER_W + ci * CHUNK
        pltpu.sync_copy(head_r.at[pl.ds(base, CHUNK)], idx_h)
        pltpu.sync_copy(rel_r.at[pl.ds(base, CHUNK)], idx_r)
        pltpu.sync_copy(tail_r.at[pl.ds(base, CHUNK)], idx_t)
        cps = [
            pltpu.async_copy(er.at[idx_h], g_hr, sem),
            pltpu.async_copy(ei.at[idx_h], g_hi, sem),
            pltpu.async_copy(er.at[idx_t], g_tr, sem),
            pltpu.async_copy(ei.at[idx_t], g_ti, sem),
            pltpu.async_copy(rrt.at[idx_r], g_rr, sem),
            pltpu.async_copy(rit.at[idx_r], g_ri, sem),
        ]
        for cp in cps:
            cp.wait()

        def group_body(g, carry2):
            tot = jnp.zeros((L,), jnp.float32)
            for row in range(L):
                c = g * L + row
                acc = jnp.zeros((L,), jnp.float32)
                for j in range(EMBED_DIM // L):
                    sl = pl.ds(j * L, L)
                    hr = g_hr[c, sl]
                    hi = g_hi[c, sl]
                    tr = g_tr[c, sl]
                    ti = g_ti[c, sl]
                    rr = g_rr[c, sl]
                    ri = g_ri[c, sl]
                    acc = acc + rr * (hr * tr + hi * ti) + ri * (hr * ti - hi * tr)
                s = lax.reduce_sum_p.bind(acc, axes=(0,))
                tot = jnp.where(iota == row, s, tot)
            out_v[pl.ds(g * L, L)] = tot
            return carry2

        lax.fori_loop(0, CHUNK // L, group_body, 0)
        pltpu.sync_copy(out_v, out_hbm.at[pl.ds(base, CHUNK)])
        return carry

    lax.fori_loop(0, NCHUNK, chunk_body, 0)


def kernel(head, relation, tail, ent_real, ent_imag, rel_real, rel_imag):
    mesh = plsc.VectorSubcoreMesh(core_axis_name="c", subcore_axis_name="s")
    f = pl.kernel(
        _body,
        mesh=mesh,
        compiler_params=pltpu.CompilerParams(
            needs_layout_passes=False, use_tc_tiling_on_sc=False),
        out_type=jax.ShapeDtypeStruct((BATCH,), jnp.float32),
        scratch_types=[
            pltpu.VMEM((CHUNK,), jnp.int32),
            pltpu.VMEM((CHUNK,), jnp.int32),
            pltpu.VMEM((CHUNK,), jnp.int32),
            pltpu.VMEM((CHUNK, EMBED_DIM), jnp.float32),
            pltpu.VMEM((CHUNK, EMBED_DIM), jnp.float32),
            pltpu.VMEM((CHUNK, EMBED_DIM), jnp.float32),
            pltpu.VMEM((CHUNK, EMBED_DIM), jnp.float32),
            pltpu.VMEM((CHUNK, EMBED_DIM), jnp.float32),
            pltpu.VMEM((CHUNK, EMBED_DIM), jnp.float32),
            pltpu.VMEM((CHUNK,), jnp.float32),
            pltpu.SemaphoreType.DMA,
        ],
    )
    return f(head, relation, tail, ent_real, ent_imag, rel_real, rel_imag)


# speedup vs baseline: 1.4919x; 1.4919x over previous
import jax
import jax.numpy as jnp
from jax import lax
from jax.experimental import pallas as pl
from jax.experimental.pallas import tpu as pltpu
from jax.experimental.pallas import tpu_sc as plsc

NUM_ENTITIES = 1000000
NUM_RELATIONS = 1000
EMBED_DIM = 64
BATCH = 16384

NC = 2
NS = 16
NW = NC * NS
B_PER_W = BATCH // NW
CHUNK = 128
NCHUNK = B_PER_W // CHUNK
L = 16


def _body(head_r, rel_r, tail_r, er, ei, rrt, rit, out_hbm,
          idx_v, idx_h, idx_r, idx_t, g_hr, g_hi, g_tr, g_ti, g_rr, g_ri,
          out_v, sem):
    wid = lax.axis_index("s") * NC + lax.axis_index("c")
    iota = lax.iota(jnp.int32, L)

    def chunk_body(ci, carry):
        base = wid * B_PER_W + ci * CHUNK
        pltpu.sync_copy(head_r.at[pl.ds(base, CHUNK)], idx_h)
        pltpu.sync_copy(rel_r.at[pl.ds(base, CHUNK)], idx_r)
        pltpu.sync_copy(tail_r.at[pl.ds(base, CHUNK)], idx_t)

        def fetch_body(g, carry2):
            vh = idx_h[pl.ds(g * L, L)]
            vt = idx_t[pl.ds(g * L, L)]
            vr = idx_r[pl.ds(g * L, L)]
            cps = []
            for row in range(L):
                r = g * L + row
                ih = vh[row]
                it = vt[row]
                ir = vr[row]
                cps.append(pltpu.make_async_copy(er.at[ih], g_hr.at[r], sem))
                cps.append(pltpu.make_async_copy(ei.at[ih], g_hi.at[r], sem))
                cps.append(pltpu.make_async_copy(er.at[it], g_tr.at[r], sem))
                cps.append(pltpu.make_async_copy(ei.at[it], g_ti.at[r], sem))
                cps.append(pltpu.make_async_copy(rrt.at[ir], g_rr.at[r], sem))
                cps.append(pltpu.make_async_copy(rit.at[ir], g_ri.at[r], sem))
            for cp in cps:
                cp.start()
            for cp in cps:
                cp.wait()
            return carry2

        lax.fori_loop(0, CHUNK // L, fetch_body, 0)

        def group_body(g, carry2):
            tot = jnp.zeros((L,), jnp.float32)
            for row in range(L):
                c = g * L + row
                acc = jnp.zeros((L,), jnp.float32)
                for j in range(EMBED_DIM // L):
                    sl = pl.ds(j * L, L)
                    hr = g_hr[c, sl]
                    hi = g_hi[c, sl]
                    tr = g_tr[c, sl]
                    ti = g_ti[c, sl]
                    rr = g_rr[c, sl]
                    ri = g_ri[c, sl]
                    acc = acc + rr * (hr * tr + hi * ti) + ri * (hr * ti - hi * tr)
                s = lax.reduce_sum_p.bind(acc, axes=(0,))
                tot = jnp.where(iota == row, s, tot)
            out_v[pl.ds(g * L, L)] = tot
            return carry2

        lax.fori_loop(0, CHUNK // L, group_body, 0)
        pltpu.sync_copy(out_v, out_hbm.at[pl.ds(base, CHUNK)])
        return carry

    lax.fori_loop(0, NCHUNK, chunk_body, 0)


def kernel(head, relation, tail, ent_real, ent_imag, rel_real, rel_imag):
    mesh = plsc.VectorSubcoreMesh(core_axis_name="c", subcore_axis_name="s")
    f = pl.kernel(
        _body,
        mesh=mesh,
        compiler_params=pltpu.CompilerParams(
            needs_layout_passes=False, use_tc_tiling_on_sc=True),
        out_type=jax.ShapeDtypeStruct((BATCH,), jnp.float32),
        scratch_types=[
            pltpu.VMEM((CHUNK,), jnp.int32),
            pltpu.VMEM((CHUNK,), jnp.int32),
            pltpu.VMEM((CHUNK,), jnp.int32),
            pltpu.VMEM((CHUNK,), jnp.int32),
            pltpu.VMEM((CHUNK, EMBED_DIM), jnp.float32),
            pltpu.VMEM((CHUNK, EMBED_DIM), jnp.float32),
            pltpu.VMEM((CHUNK, EMBED_DIM), jnp.float32),
            pltpu.VMEM((CHUNK, EMBED_DIM), jnp.float32),
            pltpu.VMEM((CHUNK, EMBED_DIM), jnp.float32),
            pltpu.VMEM((CHUNK, EMBED_DIM), jnp.float32),
            pltpu.VMEM((CHUNK,), jnp.float32),
            pltpu.SemaphoreType.DMA,
        ],
    )
    return f(head, relation, tail, ent_real, ent_imag, rel_real, rel_imag)
